# baseline (speedup 1.0000x reference)
import functools

import jax
import jax.numpy as jnp
from jax import lax
from jax.experimental import pallas as pl
from jax.experimental.pallas import tpu as pltpu
from jax.experimental.pallas import tpu_sc as plsc

N = 10000
IN_CH = 128
HID = 16
OUT_CH = 7
E = 320000

NC = 2
NS = 16
NW = NC * NS
EPW = E // NW
K = 832
NCH = EPW // K
TAIL = EPW - NCH * K
N_PAD = 10112
RPT = N_PAD // NS
NBUF = 4
LAG = 2

_mesh = plsc.VectorSubcoreMesh(
    core_axis_name="c", subcore_axis_name="s", num_cores=NC, num_subcores=NS)
_sc_params = pltpu.CompilerParams(use_tc_tiling_on_sc=False)


def _wid():
    return lax.axis_index("s") * NC + lax.axis_index("c")


@functools.partial(
    pl.kernel,
    out_type=jax.ShapeDtypeStruct((NC, N_PAD, HID), jnp.float32),
    mesh=_mesh,
    scratch_types=[
        pltpu.VMEM((EPW,), jnp.int32),
        pltpu.VMEM((K, HID), jnp.float32),
        pltpu.VMEM_SHARED((N_PAD, HID), jnp.float32),
        pltpu.SemaphoreType.DMA,
    ],
    compiler_params=_sc_params,
)
def _deg(ei_hbm, ones_hbm, zero_hbm, out_hbm, didx_v, ones_v, acc_ref, ssem):
    c = lax.axis_index("c")
    s = lax.axis_index("s")
    w = _wid()
    r0 = s * RPT

    pltpu.sync_copy(zero_hbm, acc_ref.at[pl.ds(r0, RPT)])
    pltpu.sync_copy(ei_hbm.at[pl.ds(w * EPW, EPW)], didx_v)
    pltpu.sync_copy(ones_hbm, ones_v)
    plsc.subcore_barrier()

    def drain():
        pltpu.make_async_copy(
            ones_v, acc_ref.at[didx_v.at[pl.ds(0, K)]], ssem).wait()

    def body(j, _):
        pltpu.async_copy(
            ones_v, acc_ref.at[didx_v.at[pl.ds(j * K, K)]], ssem, add=True)

        @pl.when(j >= 4)
        def _():
            drain()
        return 0

    lax.fori_loop(0, NCH, body, 0)

    def tailw(j, _):
        drain()
        return 0

    lax.fori_loop(0, 4, tailw, 0)
    pltpu.async_copy(
        ones_v.at[pl.ds(0, TAIL)],
        acc_ref.at[didx_v.at[pl.ds(NCH * K, TAIL)]], ssem, add=True)
    pltpu.make_async_copy(
        ones_v.at[pl.ds(0, TAIL)],
        acc_ref.at[didx_v.at[pl.ds(0, TAIL)]], ssem).wait()

    plsc.subcore_barrier()
    pltpu.sync_copy(acc_ref.at[pl.ds(r0, RPT)],
                    out_hbm.at[c, pl.ds(r0, RPT)])


@functools.partial(
    pl.kernel,
    out_type=jax.ShapeDtypeStruct((NC, N_PAD, HID), jnp.float32),
    mesh=_mesh,
    scratch_types=[
        pltpu.VMEM((EPW,), jnp.int32),
        pltpu.VMEM((EPW,), jnp.int32),
        pltpu.VMEM((NBUF, K, HID), jnp.float32),
        pltpu.VMEM_SHARED((N_PAD, HID), jnp.float32),
        pltpu.SemaphoreType.DMA,
        pltpu.SemaphoreType.DMA,
    ],
    compiler_params=_sc_params,
)
def _agg(z_hbm, ei_hbm, zero_hbm, out_hbm,
         gidx_v, sidx_v, gbuf, acc_ref, gsem, ssem):
    c = lax.axis_index("c")
    s = lax.axis_index("s")
    w = _wid()
    r0 = s * RPT

    pltpu.sync_copy(zero_hbm, acc_ref.at[pl.ds(r0, RPT)])
    pltpu.sync_copy(ei_hbm.at[pl.ds(w * EPW, EPW)], gidx_v)
    pltpu.sync_copy(ei_hbm.at[pl.ds(E + w * EPW, EPW)], sidx_v)
    plsc.subcore_barrier()

    def start_gather(j, b):
        pltpu.async_copy(
            z_hbm.at[gidx_v.at[pl.ds(j * K, K)]], gbuf.at[b], gsem)

    def wait_gather():
        pltpu.make_async_copy(
            z_hbm.at[gidx_v.at[pl.ds(0, K)]], gbuf.at[0], gsem).wait()

    def start_scatter(j, b):
        pltpu.async_copy(
            gbuf.at[b], acc_ref.at[sidx_v.at[pl.ds(j * K, K)]], ssem, add=True)

    def wait_scatter():
        pltpu.make_async_copy(
            gbuf.at[0], acc_ref.at[sidx_v.at[pl.ds(0, K)]], ssem).wait()

    for g in range(NBUF - LAG):
        start_gather(g, g)

    def body(j, _):
        wait_gather()

        @pl.when(j >= LAG)
        def _():
            wait_scatter()

        g = j + NBUF - LAG

        @pl.when(g < NCH)
        def _():
            start_gather(g, lax.rem(g, NBUF))

        start_scatter(j, lax.rem(j, NBUF))
        return 0

    lax.fori_loop(0, NCH, body, 0)

    def tailw(j, _):
        wait_scatter()
        return 0

    lax.fori_loop(0, LAG, tailw, 0)

    tb = gbuf.at[0, pl.ds(0, TAIL)]
    pltpu.async_copy(
        z_hbm.at[gidx_v.at[pl.ds(NCH * K, TAIL)]], tb, gsem).wait()
    pltpu.async_copy(
        tb, acc_ref.at[sidx_v.at[pl.ds(NCH * K, TAIL)]], ssem, add=True)
    pltpu.make_async_copy(
        tb, acc_ref.at[sidx_v.at[pl.ds(0, TAIL)]], ssem).wait()

    plsc.subcore_barrier()
    pltpu.sync_copy(acc_ref.at[pl.ds(r0, RPT)],
                    out_hbm.at[c, pl.ds(r0, RPT)])


FR = N_PAD * HID // 128
FRN = N * HID // 128


def _lin1_body(x3_ref, w_ref, degp_ref, z_ref, d_ref):
    df = lax.rsqrt(degp_ref[0] + degp_ref[1] + 1.0)
    w = w_ref[...]
    yf = jnp.concatenate(
        [jnp.dot(x3_ref[:, a, :], w, preferred_element_type=jnp.float32)
         for a in range(8)], axis=1)
    z_ref[...] = jnp.pad(yf * df[:FRN], ((0, FR - FRN), (0, 0)))
    d_ref[...] = df


def _lin2_body(aggp_ref, z1_ref, d_ref, wb_ref, z2_ref):
    df = d_ref[...]
    hf = jnp.maximum(df * (aggp_ref[0] + aggp_ref[1] + z1_ref[...]), 0.0)
    z2_ref[...] = jnp.dot(hf, wb_ref[...],
                          preferred_element_type=jnp.float32) * df


def _fin_body(aggp_ref, z2_ref, d_ref, out_ref):
    df = d_ref[:FRN]
    hf = df * (aggp_ref[0, :FRN] + aggp_ref[1, :FRN] + z2_ref[:FRN])
    lane = lax.broadcasted_iota(jnp.int32, (FRN, 128), 1)
    hm = jnp.where(lane % HID < OUT_CH, hf, -1e30)
    m = jnp.max(hm, axis=1, keepdims=True)
    e = jnp.exp(hm - m)
    li = lax.broadcasted_iota(jnp.int32, (128, 128), 0)
    lj = lax.broadcasted_iota(jnp.int32, (128, 128), 1)
    ones_blk = (li // HID == lj // HID).astype(jnp.float32)
    ssum = jnp.dot(e, ones_blk, precision=lax.Precision.HIGHEST,
                   preferred_element_type=jnp.float32)
    out_ref[...] = hf - (jnp.log(ssum) + m)


def _flat(a):
    return a.reshape(a.shape[0], FR, 128)


def kernel(x, edge_index, W1, W2):
    ei = edge_index.astype(jnp.int32).reshape(2 * E)
    x3 = x.reshape(FRN, 8, IN_CH)
    w2b = jnp.kron(jnp.eye(8, dtype=jnp.float32),
                   jnp.pad(W2, ((0, 0), (0, HID - OUT_CH))))

    ones = jnp.ones((K, HID), jnp.float32)
    zero = jnp.zeros((RPT, HID), jnp.float32)

    degp = _flat(_deg(ei, ones, zero))

    z1f, df = pl.pallas_call(
        _lin1_body,
        out_shape=(
            jax.ShapeDtypeStruct((FR, 128), jnp.float32),
            jax.ShapeDtypeStruct((FR, 128), jnp.float32),
        ),
    )(x3, W1, degp)

    aggp1 = _flat(_agg(z1f.reshape(N_PAD, HID), ei, zero))

    z2f = pl.pallas_call(
        _lin2_body,
        out_shape=jax.ShapeDtypeStruct((FR, 128), jnp.float32),
    )(aggp1, z1f, df, w2b)

    aggp2 = _flat(_agg(z2f.reshape(N_PAD, HID), ei, zero))

    res = pl.pallas_call(
        _fin_body,
        out_shape=jax.ShapeDtypeStruct((FRN, 128), jnp.float32),
    )(aggp2, z2f, df)

    return res.reshape(N, HID)[:, :OUT_CH]


# speedup vs baseline: 1.0986x; 1.0986x over previous
import functools

import jax
import jax.numpy as jnp
from jax import lax
from jax.experimental import pallas as pl
from jax.experimental.pallas import tpu as pltpu
from jax.experimental.pallas import tpu_sc as plsc

N = 10000
IN_CH = 128
HID = 16
OUT_CH = 7
E = 320000

NC = 2
NS = 16
NW = NC * NS
EPW = E // NW
K = 128
NCH = EPW // K
TAIL = EPW - NCH * K
N_PAD = 10112
RPT = N_PAD // NS
NBUF = 32
LAG = 16

_mesh = plsc.VectorSubcoreMesh(
    core_axis_name="c", subcore_axis_name="s", num_cores=NC, num_subcores=NS)
_sc_params = pltpu.CompilerParams(use_tc_tiling_on_sc=False)


def _wid():
    return lax.axis_index("s") * NC + lax.axis_index("c")


@functools.partial(
    pl.kernel,
    out_type=jax.ShapeDtypeStruct((NC, N_PAD, HID), jnp.float32),
    mesh=_mesh,
    scratch_types=[
        pltpu.VMEM((EPW,), jnp.int32),
        pltpu.VMEM((K, HID), jnp.float32),
        pltpu.VMEM_SHARED((N_PAD, HID), jnp.float32),
        pltpu.SemaphoreType.DMA,
    ],
    compiler_params=_sc_params,
)
def _deg(ei_hbm, ones_hbm, zero_hbm, out_hbm, didx_v, ones_v, acc_ref, ssem):
    c = lax.axis_index("c")
    s = lax.axis_index("s")
    w = _wid()
    r0 = s * RPT

    pltpu.sync_copy(zero_hbm, acc_ref.at[pl.ds(r0, RPT)])
    pltpu.sync_copy(ei_hbm.at[pl.ds(w * EPW, EPW)], didx_v)
    pltpu.sync_copy(ones_hbm, ones_v)
    plsc.subcore_barrier()

    def drain():
        pltpu.make_async_copy(
            ones_v, acc_ref.at[didx_v.at[pl.ds(0, K)]], ssem).wait()

    def body(j, _):
        pltpu.async_copy(
            ones_v, acc_ref.at[didx_v.at[pl.ds(j * K, K)]], ssem, add=True)

        @pl.when(j >= 24)
        def _():
            drain()
        return 0

    lax.fori_loop(0, NCH, body, 0)

    def tailw(j, _):
        drain()
        return 0

    lax.fori_loop(0, 24, tailw, 0)
    pltpu.async_copy(
        ones_v.at[pl.ds(0, TAIL)],
        acc_ref.at[didx_v.at[pl.ds(NCH * K, TAIL)]], ssem, add=True)
    pltpu.make_async_copy(
        ones_v.at[pl.ds(0, TAIL)],
        acc_ref.at[didx_v.at[pl.ds(0, TAIL)]], ssem).wait()

    plsc.subcore_barrier()
    pltpu.sync_copy(acc_ref.at[pl.ds(r0, RPT)],
                    out_hbm.at[c, pl.ds(r0, RPT)])


@functools.partial(
    pl.kernel,
    out_type=jax.ShapeDtypeStruct((NC, N_PAD, HID), jnp.float32),
    mesh=_mesh,
    scratch_types=[
        pltpu.VMEM((EPW,), jnp.int32),
        pltpu.VMEM((EPW,), jnp.int32),
        pltpu.VMEM((NBUF, K, HID), jnp.float32),
        pltpu.VMEM_SHARED((N_PAD, HID), jnp.float32),
        pltpu.SemaphoreType.DMA,
        pltpu.SemaphoreType.DMA,
    ],
    compiler_params=_sc_params,
)
def _agg(z_hbm, ei_hbm, zero_hbm, out_hbm,
         gidx_v, sidx_v, gbuf, acc_ref, gsem, ssem):
    c = lax.axis_index("c")
    s = lax.axis_index("s")
    w = _wid()
    r0 = s * RPT

    pltpu.sync_copy(zero_hbm, acc_ref.at[pl.ds(r0, RPT)])
    pltpu.sync_copy(ei_hbm.at[pl.ds(w * EPW, EPW)], gidx_v)
    pltpu.sync_copy(ei_hbm.at[pl.ds(E + w * EPW, EPW)], sidx_v)
    plsc.subcore_barrier()

    def start_gather(j, b):
        pltpu.async_copy(
            z_hbm.at[gidx_v.at[pl.ds(j * K, K)]], gbuf.at[b], gsem)

    def wait_gather():
        pltpu.make_async_copy(
            z_hbm.at[gidx_v.at[pl.ds(0, K)]], gbuf.at[0], gsem).wait()

    def start_scatter(j, b):
        pltpu.async_copy(
            gbuf.at[b], acc_ref.at[sidx_v.at[pl.ds(j * K, K)]], ssem, add=True)

    def wait_scatter():
        pltpu.make_async_copy(
            gbuf.at[0], acc_ref.at[sidx_v.at[pl.ds(0, K)]], ssem).wait()

    for g in range(NBUF - LAG):
        start_gather(g, g)

    def body(j, _):
        wait_gather()

        @pl.when(j >= LAG)
        def _():
            wait_scatter()

        g = j + NBUF - LAG

        @pl.when(g < NCH)
        def _():
            start_gather(g, lax.rem(g, NBUF))

        start_scatter(j, lax.rem(j, NBUF))
        return 0

    lax.fori_loop(0, NCH, body, 0)

    def tailw(j, _):
        wait_scatter()
        return 0

    lax.fori_loop(0, LAG, tailw, 0)

    tb = gbuf.at[0, pl.ds(0, TAIL)]
    pltpu.async_copy(
        z_hbm.at[gidx_v.at[pl.ds(NCH * K, TAIL)]], tb, gsem).wait()
    pltpu.async_copy(
        tb, acc_ref.at[sidx_v.at[pl.ds(NCH * K, TAIL)]], ssem, add=True)
    pltpu.make_async_copy(
        tb, acc_ref.at[sidx_v.at[pl.ds(0, TAIL)]], ssem).wait()

    plsc.subcore_barrier()
    pltpu.sync_copy(acc_ref.at[pl.ds(r0, RPT)],
                    out_hbm.at[c, pl.ds(r0, RPT)])


FR = N_PAD * HID // 128
FRN = N * HID // 128


def _mm1_body(x3_ref, w_ref, y_ref):
    w = w_ref[...]
    y_ref[...] = jnp.concatenate(
        [jnp.dot(x3_ref[:, a, :], w, preferred_element_type=jnp.float32)
         for a in range(8)], axis=1)


def _scale1_body(y_ref, degp_ref, z_ref, d_ref):
    df = lax.rsqrt(degp_ref[0] + degp_ref[1] + 1.0)
    z_ref[...] = jnp.pad(y_ref[...] * df[:FRN], ((0, FR - FRN), (0, 0)))
    d_ref[...] = df


def _lin2_body(aggp_ref, z1_ref, d_ref, wb_ref, z2_ref):
    df = d_ref[...]
    hf = jnp.maximum(df * (aggp_ref[0] + aggp_ref[1] + z1_ref[...]), 0.0)
    z2_ref[...] = jnp.dot(hf, wb_ref[...],
                          preferred_element_type=jnp.float32) * df


def _fin_body(aggp_ref, z2_ref, d_ref, out_ref):
    df = d_ref[:FRN]
    hf = df * (aggp_ref[0, :FRN] + aggp_ref[1, :FRN] + z2_ref[:FRN])
    lane = lax.broadcasted_iota(jnp.int32, (FRN, 128), 1)
    hm = jnp.where(lane % HID < OUT_CH, hf, -1e30)
    m = jnp.max(hm, axis=1, keepdims=True)
    e = jnp.exp(hm - m)
    li = lax.broadcasted_iota(jnp.int32, (128, 128), 0)
    lj = lax.broadcasted_iota(jnp.int32, (128, 128), 1)
    ones_blk = (li // HID == lj // HID).astype(jnp.float32)
    ssum = jnp.dot(e, ones_blk, precision=lax.Precision.HIGHEST,
                   preferred_element_type=jnp.float32)
    out_ref[...] = hf - (jnp.log(ssum) + m)


def _flat(a):
    return a.reshape(a.shape[0], FR, 128)


def kernel(x, edge_index, W1, W2):
    ei = edge_index.astype(jnp.int32).reshape(2 * E)
    x3 = x.reshape(FRN, 8, IN_CH)
    w2b = jnp.kron(jnp.eye(8, dtype=jnp.float32),
                   jnp.pad(W2, ((0, 0), (0, HID - OUT_CH))))

    ones = jnp.ones((K, HID), jnp.float32)
    zero = jnp.zeros((RPT, HID), jnp.float32)

    yf = pl.pallas_call(
        _mm1_body,
        out_shape=jax.ShapeDtypeStruct((FRN, 128), jnp.float32),
    )(x3, W1)

    degp = _flat(_deg(ei, ones, zero))

    z1f, df = pl.pallas_call(
        _scale1_body,
        out_shape=(
            jax.ShapeDtypeStruct((FR, 128), jnp.float32),
            jax.ShapeDtypeStruct((FR, 128), jnp.float32),
        ),
    )(yf, degp)

    aggp1 = _flat(_agg(z1f.reshape(N_PAD, HID), ei, zero))

    z2f = pl.pallas_call(
        _lin2_body,
        out_shape=jax.ShapeDtypeStruct((FR, 128), jnp.float32),
    )(aggp1, z1f, df, w2b)

    aggp2 = _flat(_agg(z2f.reshape(N_PAD, HID), ei, zero))

    res = pl.pallas_call(
        _fin_body,
        out_shape=jax.ShapeDtypeStruct((FRN, 128), jnp.float32),
    )(aggp2, z2f, df)

    return res.reshape(N, HID)[:, :OUT_CH]
